# Initial kernel scaffold; baseline (speedup 1.0000x reference)
import jax
import jax.numpy as jnp
from jax.experimental import pallas as pl


def kernel(x, pos_emb_table):
    raise NotImplementedError("write your pallas kernel here")


# speedup vs baseline: 1.6813x; 1.6813x over previous
import jax
import jax.numpy as jnp
from jax.experimental import pallas as pl


_CHUNK = 512


def _body(x_ref, t_ref, o_ref):
    o_ref[...] = x_ref[...] + t_ref[...]


def kernel(x, pos_emb_table):
    B, L, D = x.shape
    C = _CHUNK
    grid = (L // C, B)
    return pl.pallas_call(
        _body,
        grid=grid,
        in_specs=[
            pl.BlockSpec((1, C, D), lambda i, b: (b, i, 0)),
            pl.BlockSpec((1, C, D), lambda i, b: (0, i, 0)),
        ],
        out_specs=pl.BlockSpec((1, C, D), lambda i, b: (b, i, 0)),
        out_shape=jax.ShapeDtypeStruct((B, L, D), x.dtype),
    )(x, pos_emb_table[None])


# speedup vs baseline: 1.8797x; 1.1180x over previous
import jax
import jax.numpy as jnp
from jax.experimental import pallas as pl


_CHUNK = 1024


def _body(x_ref, t_ref, o_ref):
    o_ref[...] = x_ref[...] + t_ref[...]


def kernel(x, pos_emb_table):
    B, L, D = x.shape
    C = _CHUNK
    grid = (L // C, B)
    return pl.pallas_call(
        _body,
        grid=grid,
        in_specs=[
            pl.BlockSpec((1, C, D), lambda i, b: (b, i, 0)),
            pl.BlockSpec((1, C, D), lambda i, b: (0, i, 0)),
        ],
        out_specs=pl.BlockSpec((1, C, D), lambda i, b: (b, i, 0)),
        out_shape=jax.ShapeDtypeStruct((B, L, D), x.dtype),
    )(x, pos_emb_table[None])


# speedup vs baseline: 1.9956x; 1.0616x over previous
import jax
import jax.numpy as jnp
from jax.experimental import pallas as pl


_CHUNK = 2048


def _body(x_ref, t_ref, o_ref):
    o_ref[...] = x_ref[...] + t_ref[...]


def kernel(x, pos_emb_table):
    B, L, D = x.shape
    C = _CHUNK
    grid = (L // C, B)
    return pl.pallas_call(
        _body,
        grid=grid,
        in_specs=[
            pl.BlockSpec((1, C, D), lambda i, b: (b, i, 0)),
            pl.BlockSpec((1, C, D), lambda i, b: (0, i, 0)),
        ],
        out_specs=pl.BlockSpec((1, C, D), lambda i, b: (b, i, 0)),
        out_shape=jax.ShapeDtypeStruct((B, L, D), x.dtype),
    )(x, pos_emb_table[None])
